# baseline (speedup 1.0000x reference)
import functools

import jax
import jax.numpy as jnp
from jax.experimental import pallas as pl
from jax.experimental.pallas import tpu as pltpu

_K = 128
_QBLK = 64
_CBLK = 2048


def _body(nchunks, npad, a_ref, b_ref, a2_ref, b2_ref, lab_ref, o_ref, bits_ref):
    c = pl.program_id(1)
    ab = jax.lax.dot_general(
        a_ref[...], b_ref[...], (((1,), (1,)), ((), ())),
        preferred_element_type=jnp.float32)
    d2 = a2_ref[...] + b2_ref[...] - 2.0 * ab
    dist = jnp.sqrt(jnp.maximum(d2, 0.0))
    bits_ref[:, pl.ds(c * _CBLK, _CBLK)] = jax.lax.bitcast_convert_type(
        dist, jnp.int32)

    @pl.when(c == nchunks - 1)
    def _select():
        bits = bits_ref[...]
        kk = jnp.int32(_K)

        def dstep(_, lohi):
            lo, hi = lohi
            mid = lo + (hi - lo) // 2
            cnt = jnp.sum((bits <= mid).astype(jnp.int32), axis=1,
                          keepdims=True)
            ge = cnt >= kk
            return (jnp.where(ge, lo, mid), jnp.where(ge, mid, hi))

        lo0 = jnp.full((_QBLK, 1), -1, jnp.int32)
        hi0 = jnp.full((_QBLK, 1), 0x7F800000, jnp.int32)
        _, d128 = jax.lax.fori_loop(0, 31, dstep, (lo0, hi0))

        lt = bits < d128
        eq = bits == d128
        lab = lab_ref[...]
        c_lt = jnp.sum(lt.astype(jnp.int32), axis=1, keepdims=True)
        m = kk - c_lt
        c1_lt = jnp.sum(jnp.where(lt, lab, 0.0), axis=1, keepdims=True)

        idx = jax.lax.broadcasted_iota(jnp.int32, (_QBLK, npad), 1)

        def istep(_, lohi):
            lo, hi = lohi
            mid = lo + (hi - lo) // 2
            cnt = jnp.sum((eq & (idx <= mid)).astype(jnp.int32), axis=1,
                          keepdims=True)
            ge = cnt >= m
            return (jnp.where(ge, lo, mid), jnp.where(ge, mid, hi))

        ilo0 = jnp.full((_QBLK, 1), -1, jnp.int32)
        ihi0 = jnp.full((_QBLK, 1), npad - 1, jnp.int32)
        _, isel = jax.lax.fori_loop(0, 17, istep, (ilo0, ihi0))

        c1_eq = jnp.sum(jnp.where(eq & (idx <= isel), lab, 0.0), axis=1,
                        keepdims=True)
        votes1 = c1_lt + c1_eq
        pred = (votes1 * 2.0 > jnp.float32(_K)).astype(jnp.int32)
        o_ref[...] = pred.reshape(1, 1, _QBLK)


@jax.jit
def kernel(input, data, labels):
    q, d_feat = input.shape
    n = data.shape[0]
    nchunks = -(-n // _CBLK)
    npad = nchunks * _CBLK
    nqb = q // _QBLK

    a2 = jnp.sum(input * input, axis=1, keepdims=True)
    b2 = jnp.sum(data * data, axis=1)
    b2p = jnp.full((npad,), jnp.inf, jnp.float32).at[:n].set(b2)[None, :]
    datap = jnp.zeros((npad, d_feat), jnp.float32).at[:n].set(data)
    labp = jnp.zeros((npad,), jnp.float32).at[:n].set(labels)[None, :]

    out = pl.pallas_call(
        functools.partial(_body, nchunks, npad),
        grid=(nqb, nchunks),
        in_specs=[
            pl.BlockSpec((_QBLK, d_feat), lambda qb, c: (qb, 0)),
            pl.BlockSpec((_CBLK, d_feat), lambda qb, c: (c, 0)),
            pl.BlockSpec((_QBLK, 1), lambda qb, c: (qb, 0)),
            pl.BlockSpec((1, _CBLK), lambda qb, c: (0, c)),
            pl.BlockSpec((1, npad), lambda qb, c: (0, 0)),
        ],
        out_specs=pl.BlockSpec((1, 1, _QBLK), lambda qb, c: (qb, 0, 0)),
        out_shape=jax.ShapeDtypeStruct((nqb, 1, _QBLK), jnp.int32),
        scratch_shapes=[pltpu.VMEM((_QBLK, npad), jnp.int32)],
    )(input, datap, a2, b2p, labp)
    return (out.reshape(q), 0)
